# baseline (speedup 1.0000x reference)
import jax
import jax.numpy as jnp
from jax import lax
from jax.experimental import pallas as pl
from jax.experimental.pallas import tpu as pltpu
from jax.experimental.pallas import tpu_sc as plsc

N = 10000
E = 320000
F = 128

NC = 2
NS = 16
NW = NC * NS

CHUNK = 128
EPW = E // NW
NCH = 80
EPW_PAD = NCH * CHUNK
G = 16
NG = NCH // G
NPAD = 10240
ROWS_PER_TILE = NPAD // NS
DUMMY_ROW = N + 100

_f32 = jnp.float32


def _sc_segment_sum(feat, src_r, dst_r, zrow, with_counts, zcnt=None, ones=None):
  mesh = plsc.VectorSubcoreMesh(core_axis_name="c", subcore_axis_name="s")

  out_type = [jax.ShapeDtypeStruct((NC, NPAD, F), _f32)]
  scratch = [
      pltpu.VMEM((G, CHUNK), jnp.int32),
      pltpu.VMEM((G, CHUNK), jnp.int32),
      pltpu.VMEM((CHUNK, F), _f32),
      pltpu.VMEM((CHUNK, F), _f32),
      pltpu.VMEM_SHARED((NPAD, F), _f32),
      pltpu.SemaphoreType.DMA,
      pltpu.SemaphoreType.DMA,
      pltpu.SemaphoreType.DMA,
  ]
  if with_counts:
    out_type.append(jax.ShapeDtypeStruct((NC, NPAD), _f32))
    scratch += [
        pltpu.VMEM((CHUNK,), _f32),
        pltpu.VMEM_SHARED((NPAD,), _f32),
        pltpu.SemaphoreType.DMA,
    ]

  def body(*refs):
    if with_counts:
      (feat_h, src_h, dst_h, zrow_h, zcnt_h, ones_h, agg_o, cnt_o,
       src_v, dst_v, rows0, rows1, agg_sh, sem_i, sem_g, sem_s,
       ones_v, cnt_sh, sem_c) = refs
    else:
      (feat_h, src_h, dst_h, zrow_h, agg_o,
       src_v, dst_v, rows0, rows1, agg_sh, sem_i, sem_g, sem_s) = refs
    bufs = (rows0, rows1)

    cid = lax.axis_index("c")
    sid = lax.axis_index("s")
    wid = sid * NC + cid
    row0 = sid * ROWS_PER_TILE

    pltpu.sync_copy(zrow_h, agg_sh.at[pl.ds(row0, ROWS_PER_TILE)])
    if with_counts:
      pltpu.sync_copy(zcnt_h, cnt_sh.at[pl.ds(row0, ROWS_PER_TILE)])
      pltpu.sync_copy(ones_h, ones_v)
    plsc.subcore_barrier()

    def outer(g, carry):
      di1 = pltpu.async_copy(src_h.at[wid, pl.ds(g * G, G)], src_v, sem_i)
      di2 = pltpu.async_copy(dst_h.at[wid, pl.ds(g * G, G)], dst_v, sem_i)
      di1.wait()
      di2.wait()

      def gather2(j):
        return (
            pltpu.async_copy(feat_h.at[src_v.at[j, pl.ds(0, 64)]],
                             bufs[j % 2].at[pl.ds(0, 64)], sem_g),
            pltpu.async_copy(feat_h.at[src_v.at[j, pl.ds(64, 64)]],
                             bufs[j % 2].at[pl.ds(64, 64)], sem_g),
        )

      g_desc = [None] * G
      s_desc = [None] * G
      c_desc = [None] * G
      g_desc[0] = gather2(0)
      for j in range(G):
        if j >= 1:
          s_desc[j - 1].wait()
        if j + 1 < G:
          g_desc[j + 1] = gather2(j + 1)
        g_desc[j][0].wait()
        g_desc[j][1].wait()
        s_desc[j] = pltpu.async_copy(
            bufs[j % 2], agg_sh.at[dst_v.at[j]], sem_s, add=True)
        if with_counts:
          c_desc[j] = pltpu.async_copy(
              ones_v, cnt_sh.at[dst_v.at[j]], sem_c, add=True)
      s_desc[G - 1].wait()
      if with_counts:
        for j in range(G):
          c_desc[j].wait()
      return carry

    lax.fori_loop(0, NG, outer, 0)
    plsc.subcore_barrier()

    pltpu.sync_copy(agg_sh.at[pl.ds(row0, ROWS_PER_TILE)],
                    agg_o.at[cid, pl.ds(row0, ROWS_PER_TILE)])
    if with_counts:
      pltpu.sync_copy(cnt_sh.at[pl.ds(row0, ROWS_PER_TILE)],
                      cnt_o.at[cid, pl.ds(row0, ROWS_PER_TILE)])

  k = pl.kernel(body, out_type=tuple(out_type), mesh=mesh,
                scratch_types=tuple(scratch))
  if with_counts:
    return k(feat, src_r, dst_r, zrow, zcnt, ones)
  return k(feat, src_r, dst_r, zrow)[0]


def _dense_layer(agg2, cnt2, x, w_rel, w_root, b):
  def body(agg_ref, cnt_ref, x_ref, wrel_ref, wroot_ref, b_ref, o_ref):
    aggs = agg_ref[0, :N, :] + agg_ref[1, :N, :]
    cnt = cnt_ref[0, :N, :] + cnt_ref[1, :N, :]
    scaled = aggs / jnp.maximum(cnt, 1.0)
    h = (jnp.dot(scaled, wrel_ref[...], preferred_element_type=_f32)
         + jnp.dot(x_ref[...], wroot_ref[...], preferred_element_type=_f32)
         + b_ref[...])
    o_ref[...] = jnp.maximum(h, 0.0)

  return pl.pallas_call(
      body,
      out_shape=jax.ShapeDtypeStruct((N, F), _f32),
  )(agg2, cnt2, x, w_rel, w_root, b)


def _head(h, wh, bh, wc, bc, wo, bo):
  V = wo.shape[1]

  def body(h_ref, wh_ref, bh_ref, wc_ref, bc_ref, wo_ref, bo_ref,
           logits_ref, hidden_ref, cell_ref):
    pooled = jnp.mean(h_ref[...], axis=0, keepdims=True)
    hidden = jnp.dot(pooled, wh_ref[...], preferred_element_type=_f32) + bh_ref[...]
    cell = jnp.dot(pooled, wc_ref[...], preferred_element_type=_f32) + bc_ref[...]
    y = jnp.dot(hidden, wo_ref[...], preferred_element_type=_f32) + bo_ref[...]
    m = jnp.max(y, axis=1, keepdims=True)
    z = y - m
    lse = jnp.log(jnp.sum(jnp.exp(z), axis=1, keepdims=True))
    logits_ref[...] = z - lse
    hidden_ref[...] = hidden
    cell_ref[...] = cell

  return pl.pallas_call(
      body,
      out_shape=(
          jax.ShapeDtypeStruct((1, V), _f32),
          jax.ShapeDtypeStruct((1, h.shape[1]), _f32),
          jax.ShapeDtypeStruct((1, h.shape[1]), _f32),
      ),
  )(h, wh, bh, wc, bc, wo, bo)


def kernel(prev_symbol, x, edge_index, W1_rel, W1_root, b1,
           W2_rel, W2_root, b2, Wh, bh, Wc, bc, Wo, bo):
  del prev_symbol

  src = edge_index[0].astype(jnp.int32).reshape(NW, EPW)
  dst = edge_index[1].astype(jnp.int32).reshape(NW, EPW)
  pad = EPW_PAD - EPW
  src_r = jnp.pad(src, ((0, 0), (0, pad))).reshape(NW, NCH, CHUNK)
  dst_r = jnp.pad(dst, ((0, 0), (0, pad)),
                  constant_values=DUMMY_ROW).reshape(NW, NCH, CHUNK)

  zrow = jnp.zeros((ROWS_PER_TILE, F), _f32)
  zcnt = jnp.zeros((ROWS_PER_TILE,), _f32)
  ones = jnp.ones((CHUNK,), _f32)
  b1r = b1.reshape(1, F)
  b2r = b2.reshape(1, F)
  bhr = bh.reshape(1, -1)
  bcr = bc.reshape(1, -1)
  bor = bo.reshape(1, -1)

  agg1, cnt = _sc_segment_sum(x, src_r, dst_r, zrow, True, zcnt, ones)
  cnt_col = cnt.reshape(NC, NPAD, 1)
  h1 = _dense_layer(agg1, cnt_col, x, W1_rel, W1_root, b1r)
  agg2 = _sc_segment_sum(h1, src_r, dst_r, zrow, False)
  h2 = _dense_layer(agg2, cnt_col, h1, W2_rel, W2_root, b2r)
  return _head(h2, Wh, bhr, Wc, bcr, Wo, bor)


# speedup vs baseline: 1.0010x; 1.0010x over previous
import jax
import jax.numpy as jnp
from jax import lax
from jax.experimental import pallas as pl
from jax.experimental.pallas import tpu as pltpu
from jax.experimental.pallas import tpu_sc as plsc

N = 10000
E = 320000
F = 128

NC = 2
NS = 16
NW = NC * NS

CHUNK = 128
EPW = E // NW
NCH = 80
EPW_PAD = NCH * CHUNK
G = 16
NG = NCH // G
NPAD = 10240
ROWS_PER_TILE = NPAD // NS
DUMMY_ROW = N + 100

_f32 = jnp.float32


def _sc_segment_sum(feat, src_r, dst_r, zrow, with_counts, zcnt=None, ones=None):
  mesh = plsc.VectorSubcoreMesh(core_axis_name="c", subcore_axis_name="s")

  out_type = [jax.ShapeDtypeStruct((NC, NPAD, F), _f32)]
  scratch = [
      pltpu.VMEM((G, CHUNK), jnp.int32),
      pltpu.VMEM((G, CHUNK), jnp.int32),
      pltpu.VMEM((CHUNK, F), _f32),
      pltpu.VMEM((CHUNK, F), _f32),
      pltpu.VMEM_SHARED((NPAD, F), _f32),
      pltpu.SemaphoreType.DMA,
      pltpu.SemaphoreType.DMA,
      pltpu.SemaphoreType.DMA,
  ]
  if with_counts:
    out_type.append(jax.ShapeDtypeStruct((NC, NPAD), _f32))
    scratch += [
        pltpu.VMEM((CHUNK,), _f32),
        pltpu.VMEM_SHARED((NPAD,), _f32),
        pltpu.SemaphoreType.DMA,
    ]

  def body(*refs):
    if with_counts:
      (feat_h, src_h, dst_h, zrow_h, zcnt_h, ones_h, agg_o, cnt_o,
       src_v, dst_v, rows0, rows1, agg_sh, sem_i, sem_g, sem_s,
       ones_v, cnt_sh, sem_c) = refs
    else:
      (feat_h, src_h, dst_h, zrow_h, agg_o,
       src_v, dst_v, rows0, rows1, agg_sh, sem_i, sem_g, sem_s) = refs
    bufs = (rows0, rows1)

    cid = lax.axis_index("c")
    sid = lax.axis_index("s")
    wid = sid * NC + cid
    row0 = sid * ROWS_PER_TILE

    pltpu.sync_copy(zrow_h, agg_sh.at[pl.ds(row0, ROWS_PER_TILE)])
    if with_counts:
      pltpu.sync_copy(zcnt_h, cnt_sh.at[pl.ds(row0, ROWS_PER_TILE)])
      pltpu.sync_copy(ones_h, ones_v)
    plsc.subcore_barrier()

    def outer(g, carry):
      di1 = pltpu.async_copy(src_h.at[wid, pl.ds(g * G, G)], src_v, sem_i)
      di2 = pltpu.async_copy(dst_h.at[wid, pl.ds(g * G, G)], dst_v, sem_i)
      di1.wait()
      di2.wait()

      g_desc = [None] * G
      s_desc = [None] * G
      c_desc = [None] * G
      g_desc[0] = pltpu.async_copy(feat_h.at[src_v.at[0]], bufs[0], sem_g)
      for j in range(G):
        if j >= 1:
          s_desc[j - 1].wait()
        if j + 1 < G:
          g_desc[j + 1] = pltpu.async_copy(
              feat_h.at[src_v.at[j + 1]], bufs[(j + 1) % 2], sem_g)
        g_desc[j].wait()
        s_desc[j] = pltpu.async_copy(
            bufs[j % 2], agg_sh.at[dst_v.at[j]], sem_s, add=True)
        if with_counts:
          c_desc[j] = pltpu.async_copy(
              ones_v, cnt_sh.at[dst_v.at[j]], sem_c, add=True)
      s_desc[G - 1].wait()
      if with_counts:
        for j in range(G):
          c_desc[j].wait()
      return carry

    lax.fori_loop(0, NG, outer, 0)
    plsc.subcore_barrier()

    pltpu.sync_copy(agg_sh.at[pl.ds(row0, ROWS_PER_TILE)],
                    agg_o.at[cid, pl.ds(row0, ROWS_PER_TILE)])
    if with_counts:
      pltpu.sync_copy(cnt_sh.at[pl.ds(row0, ROWS_PER_TILE)],
                      cnt_o.at[cid, pl.ds(row0, ROWS_PER_TILE)])

  k = pl.kernel(body, out_type=tuple(out_type), mesh=mesh,
                scratch_types=tuple(scratch))
  if with_counts:
    return k(feat, src_r, dst_r, zrow, zcnt, ones)
  return k(feat, src_r, dst_r, zrow)[0]


def _dense_layer(agg2, cnt2, x, w_rel, w_root, b):
  def body(agg_ref, cnt_ref, x_ref, wrel_ref, wroot_ref, b_ref, o_ref):
    aggs = agg_ref[0, :N, :] + agg_ref[1, :N, :]
    cnt = cnt_ref[0, :N, :] + cnt_ref[1, :N, :]
    scaled = aggs / jnp.maximum(cnt, 1.0)
    h = (jnp.dot(scaled, wrel_ref[...], preferred_element_type=_f32)
         + jnp.dot(x_ref[...], wroot_ref[...], preferred_element_type=_f32)
         + b_ref[...])
    o_ref[...] = jnp.maximum(h, 0.0)

  return pl.pallas_call(
      body,
      out_shape=jax.ShapeDtypeStruct((N, F), _f32),
  )(agg2, cnt2, x, w_rel, w_root, b)


def _head(h, wh, bh, wc, bc, wo, bo):
  V = wo.shape[1]

  def body(h_ref, wh_ref, bh_ref, wc_ref, bc_ref, wo_ref, bo_ref,
           logits_ref, hidden_ref, cell_ref):
    pooled = jnp.mean(h_ref[...], axis=0, keepdims=True)
    hidden = jnp.dot(pooled, wh_ref[...], preferred_element_type=_f32) + bh_ref[...]
    cell = jnp.dot(pooled, wc_ref[...], preferred_element_type=_f32) + bc_ref[...]
    y = jnp.dot(hidden, wo_ref[...], preferred_element_type=_f32) + bo_ref[...]
    m = jnp.max(y, axis=1, keepdims=True)
    z = y - m
    lse = jnp.log(jnp.sum(jnp.exp(z), axis=1, keepdims=True))
    logits_ref[...] = z - lse
    hidden_ref[...] = hidden
    cell_ref[...] = cell

  return pl.pallas_call(
      body,
      out_shape=(
          jax.ShapeDtypeStruct((1, V), _f32),
          jax.ShapeDtypeStruct((1, h.shape[1]), _f32),
          jax.ShapeDtypeStruct((1, h.shape[1]), _f32),
      ),
  )(h, wh, bh, wc, bc, wo, bo)


def kernel(prev_symbol, x, edge_index, W1_rel, W1_root, b1,
           W2_rel, W2_root, b2, Wh, bh, Wc, bc, Wo, bo):
  del prev_symbol

  src = edge_index[0].astype(jnp.int32).reshape(NW, EPW)
  dst = edge_index[1].astype(jnp.int32).reshape(NW, EPW)
  pad = EPW_PAD - EPW
  src_r = jnp.pad(src, ((0, 0), (0, pad))).reshape(NW, NCH, CHUNK)
  dst_r = jnp.pad(dst, ((0, 0), (0, pad)),
                  constant_values=DUMMY_ROW).reshape(NW, NCH, CHUNK)

  zrow = jnp.zeros((ROWS_PER_TILE, F), _f32)
  zcnt = jnp.zeros((ROWS_PER_TILE,), _f32)
  ones = jnp.ones((CHUNK,), _f32)
  b1r = b1.reshape(1, F)
  b2r = b2.reshape(1, F)
  bhr = bh.reshape(1, -1)
  bcr = bc.reshape(1, -1)
  bor = bo.reshape(1, -1)

  agg1, cnt = _sc_segment_sum(x, src_r, dst_r, zrow, True, zcnt, ones)
  cnt_col = cnt.reshape(NC, NPAD, 1)
  h1 = _dense_layer(agg1, cnt_col, x, W1_rel, W1_root, b1r)
  agg2 = _sc_segment_sum(h1, src_r, dst_r, zrow, False)
  h2 = _dense_layer(agg2, cnt_col, h1, W2_rel, W2_root, b2r)
  return _head(h2, Wh, bhr, Wc, bcr, Wo, bor)


# speedup vs baseline: 1.0058x; 1.0048x over previous
import jax
import jax.numpy as jnp
from jax import lax
from jax.experimental import pallas as pl
from jax.experimental.pallas import tpu as pltpu
from jax.experimental.pallas import tpu_sc as plsc

N = 10000
E = 320000
F = 128

NC = 2
NS = 16
NW = NC * NS

CHUNK = 128
EPW = E // NW
NCH = 80
EPW_PAD = NCH * CHUNK
G = 16
NG = NCH // G
NPAD = 10240
ROWS_PER_TILE = NPAD // NS
DUMMY_ROW = N + 100

_f32 = jnp.float32


def _sc_segment_sum(feat, src_r, dst_r, zrow, with_counts, zcnt=None, ones=None):
  mesh = plsc.VectorSubcoreMesh(core_axis_name="c", subcore_axis_name="s")

  out_type = [jax.ShapeDtypeStruct((NC, NPAD, F), _f32)]
  scratch = [
      pltpu.VMEM((G, CHUNK), jnp.int32),
      pltpu.VMEM((G, CHUNK), jnp.int32),
      pltpu.VMEM((CHUNK, F), _f32),
      pltpu.VMEM((CHUNK, F), _f32),
      pltpu.VMEM_SHARED((NPAD, F), _f32),
      pltpu.SemaphoreType.DMA,
      pltpu.SemaphoreType.DMA,
      pltpu.SemaphoreType.DMA,
  ]
  if with_counts:
    out_type.append(jax.ShapeDtypeStruct((NC, NPAD), _f32))
    scratch += [
        pltpu.VMEM((CHUNK,), _f32),
        pltpu.VMEM_SHARED((NPAD,), _f32),
        pltpu.SemaphoreType.DMA,
    ]

  def body(*refs):
    if with_counts:
      (feat_h, src_h, dst_h, zrow_h, zcnt_h, ones_h, agg_o, cnt_o,
       src_v, dst_v, rows0, rows1, agg_sh, sem_i, sem_g, sem_s,
       ones_v, cnt_sh, sem_c) = refs
    else:
      (feat_h, src_h, dst_h, zrow_h, agg_o,
       src_v, dst_v, rows0, rows1, agg_sh, sem_i, sem_g, sem_s) = refs
    bufs = (rows0, rows1)

    cid = lax.axis_index("c")
    sid = lax.axis_index("s")
    wid = sid * NC + cid
    row0 = sid * ROWS_PER_TILE

    pltpu.sync_copy(zrow_h, agg_sh.at[pl.ds(row0, ROWS_PER_TILE)])
    if with_counts:
      pltpu.sync_copy(zcnt_h, cnt_sh.at[pl.ds(row0, ROWS_PER_TILE)])
      pltpu.sync_copy(ones_h, ones_v)
    plsc.subcore_barrier()

    def outer(g, carry):
      di1 = pltpu.async_copy(src_h.at[wid, pl.ds(g * G, G)], src_v, sem_i)
      di2 = pltpu.async_copy(dst_h.at[wid, pl.ds(g * G, G)], dst_v, sem_i)
      di1.wait()
      di2.wait()

      g_desc = [None] * G
      s_desc = [None] * G
      c_desc = [None] * G
      g_desc[0] = pltpu.async_copy(feat_h.at[src_v.at[0]], bufs[0], sem_g)
      for j in range(G):
        if j >= 1:
          s_desc[j - 1].wait()
        if j + 1 < G:
          g_desc[j + 1] = pltpu.async_copy(
              feat_h.at[src_v.at[j + 1]], bufs[(j + 1) % 2], sem_g)
        g_desc[j].wait()
        s_desc[j] = pltpu.async_copy(
            bufs[j % 2], agg_sh.at[dst_v.at[j]], sem_s, add=True)
        if with_counts:
          c_desc[j] = pltpu.async_copy(
              ones_v, cnt_sh.at[dst_v.at[j]], sem_c, add=True)
      s_desc[G - 1].wait()
      if with_counts:
        for j in range(G):
          c_desc[j].wait()
      return carry

    lax.fori_loop(0, NG, outer, 0)
    plsc.subcore_barrier()

    pltpu.sync_copy(agg_sh.at[pl.ds(row0, ROWS_PER_TILE)],
                    agg_o.at[cid, pl.ds(row0, ROWS_PER_TILE)])
    if with_counts:
      pltpu.sync_copy(cnt_sh.at[pl.ds(row0, ROWS_PER_TILE)],
                      cnt_o.at[cid, pl.ds(row0, ROWS_PER_TILE)])

  k = pl.kernel(body, out_type=tuple(out_type), mesh=mesh,
                scratch_types=tuple(scratch))
  if with_counts:
    return k(feat, src_r, dst_r, zrow, zcnt, ones)
  return k(feat, src_r, dst_r, zrow)[0]


def _dense_layer(agg2, cnt2, x, w_rel, w_root, b):
  def body(agg_ref, cnt_ref, x_ref, wrel_ref, wroot_ref, b_ref, o_ref):
    aggs = agg_ref[0, :N, :] + agg_ref[1, :N, :]
    cnt = cnt_ref[0, :N, :] + cnt_ref[1, :N, :]
    scaled = aggs / jnp.maximum(cnt, 1.0)
    h = (jnp.dot(scaled, wrel_ref[...], preferred_element_type=_f32)
         + jnp.dot(x_ref[...], wroot_ref[...], preferred_element_type=_f32)
         + b_ref[...])
    o_ref[...] = jnp.maximum(h, 0.0)

  return pl.pallas_call(
      body,
      out_shape=jax.ShapeDtypeStruct((N, F), _f32),
  )(agg2, cnt2, x, w_rel, w_root, b)


def _dense_head(agg2, cnt2, h1, w_rel, w_root, b, wh, bh, wc, bc, wo, bo):
  V = wo.shape[1]

  def body(agg_ref, cnt_ref, x_ref, wrel_ref, wroot_ref, b_ref,
           wh_ref, bh_ref, wc_ref, bc_ref, wo_ref, bo_ref,
           logits_ref, hidden_ref, cell_ref):
    aggs = agg_ref[0, :N, :] + agg_ref[1, :N, :]
    cnt = cnt_ref[0, :N, :] + cnt_ref[1, :N, :]
    scaled = aggs / jnp.maximum(cnt, 1.0)
    h = (jnp.dot(scaled, wrel_ref[...], preferred_element_type=_f32)
         + jnp.dot(x_ref[...], wroot_ref[...], preferred_element_type=_f32)
         + b_ref[...])
    h = jnp.maximum(h, 0.0)
    pooled = jnp.mean(h, axis=0, keepdims=True)
    hidden = jnp.dot(pooled, wh_ref[...], preferred_element_type=_f32) + bh_ref[...]
    cell = jnp.dot(pooled, wc_ref[...], preferred_element_type=_f32) + bc_ref[...]
    y = jnp.dot(hidden, wo_ref[...], preferred_element_type=_f32) + bo_ref[...]
    m = jnp.max(y, axis=1, keepdims=True)
    z = y - m
    lse = jnp.log(jnp.sum(jnp.exp(z), axis=1, keepdims=True))
    logits_ref[...] = z - lse
    hidden_ref[...] = hidden
    cell_ref[...] = cell

  return pl.pallas_call(
      body,
      out_shape=(
          jax.ShapeDtypeStruct((1, V), _f32),
          jax.ShapeDtypeStruct((1, wh.shape[1]), _f32),
          jax.ShapeDtypeStruct((1, wc.shape[1]), _f32),
      ),
  )(agg2, cnt2, h1, w_rel, w_root, b, wh, bh, wc, bc, wo, bo)


def kernel(prev_symbol, x, edge_index, W1_rel, W1_root, b1,
           W2_rel, W2_root, b2, Wh, bh, Wc, bc, Wo, bo):
  del prev_symbol

  src = edge_index[0].astype(jnp.int32).reshape(NW, EPW)
  dst = edge_index[1].astype(jnp.int32).reshape(NW, EPW)
  pad = EPW_PAD - EPW
  src_r = jnp.pad(src, ((0, 0), (0, pad))).reshape(NW, NCH, CHUNK)
  dst_r = jnp.pad(dst, ((0, 0), (0, pad)),
                  constant_values=DUMMY_ROW).reshape(NW, NCH, CHUNK)

  zrow = jnp.zeros((ROWS_PER_TILE, F), _f32)
  zcnt = jnp.zeros((ROWS_PER_TILE,), _f32)
  ones = jnp.ones((CHUNK,), _f32)
  b1r = b1.reshape(1, F)
  b2r = b2.reshape(1, F)
  bhr = bh.reshape(1, -1)
  bcr = bc.reshape(1, -1)
  bor = bo.reshape(1, -1)

  agg1, cnt = _sc_segment_sum(x, src_r, dst_r, zrow, True, zcnt, ones)
  cnt_col = cnt.reshape(NC, NPAD, 1)
  h1 = _dense_layer(agg1, cnt_col, x, W1_rel, W1_root, b1r)
  agg2 = _sc_segment_sum(h1, src_r, dst_r, zrow, False)
  return _dense_head(agg2, cnt_col, h1, W2_rel, W2_root, b2r,
                     Wh, bhr, Wc, bcr, Wo, bor)
